# baseline (speedup 1.0000x reference)
import functools

import jax
import jax.numpy as jnp
from jax import lax
from jax.experimental import pallas as pl
from jax.experimental.pallas import tpu as pltpu
from jax.experimental.pallas import tpu_sc as plsc

N = 10000
E = 320000
D_FEAT = 128
D_EDGE = 16

NC = 2
NS = 16
NW = NC * NS
EPW = E // NW
CHUNK = 2000
NCHUNK = EPW // CHUNK
NPAD = 10240
ROWS_PER_TILE = NPAD // NS

_mesh = plsc.VectorSubcoreMesh(
    core_axis_name="c", subcore_axis_name="s", num_cores=NC, num_subcores=NS
)


@functools.partial(
    pl.kernel,
    out_type=jax.ShapeDtypeStruct((NC, NPAD, D_EDGE), jnp.float32),
    mesh=_mesh,
    compiler_params=pltpu.CompilerParams(use_tc_tiling_on_sc=False),
    scratch_types=[
        pltpu.VMEM_SHARED((NPAD, D_EDGE), jnp.float32),
        pltpu.VMEM((EPW,), jnp.int32),
        pltpu.VMEM((CHUNK, D_EDGE), jnp.float32),
        pltpu.VMEM((CHUNK, D_EDGE), jnp.float32),
        pltpu.VMEM((ROWS_PER_TILE, D_EDGE), jnp.float32),
        pltpu.SemaphoreType.DMA,
        pltpu.SemaphoreType.DMA,
    ],
)
def _sc_scatter(edge_hbm, idx_hbm, out_hbm, acc, idx_v, ebufa, ebufb, zbuf, sema, semb):
    cid = lax.axis_index("c")
    sid = lax.axis_index("s")
    wid = sid * NC + cid
    ebase = wid * EPW

    @pl.loop(0, ROWS_PER_TILE)
    def _zero(i):
        zbuf[i, :] = jnp.zeros((D_EDGE,), jnp.float32)

    pltpu.sync_copy(zbuf, acc.at[pl.ds(sid * ROWS_PER_TILE, ROWS_PER_TILE)])
    plsc.subcore_barrier()

    pltpu.sync_copy(idx_hbm.at[pl.ds(ebase, EPW)], idx_v)

    bufs = (ebufa, ebufb)
    sems = (sema, semb)
    cps = [None, None]
    for j in range(NCHUNK + 1):
        if j < NCHUNK:
            cps[j % 2] = pltpu.async_copy(
                edge_hbm.at[pl.ds(ebase + j * CHUNK, CHUNK)], bufs[j % 2], sems[j % 2]
            )
        if j >= 1:
            k = j - 1
            cps[k % 2].wait()
            pltpu.sync_copy(
                bufs[k % 2], acc.at[idx_v.at[pl.ds(k * CHUNK, CHUNK)]], add=True
            )

    plsc.subcore_barrier()

    rows = pl.ds(sid * ROWS_PER_TILE, ROWS_PER_TILE)
    pltpu.sync_copy(acc.at[rows], zbuf)
    pltpu.sync_copy(zbuf, out_hbm.at[cid, rows])


_CE = 32000


def _repack_body(ea_ref, o_ref):
    t = ea_ref[...].T
    o_ref[...] = t.reshape(_CE // 8, 8, D_EDGE)


_repack = pl.pallas_call(
    _repack_body,
    grid=(E // _CE,),
    in_specs=[pl.BlockSpec((D_EDGE, _CE), lambda i: (0, i))],
    out_specs=pl.BlockSpec((_CE // 8, 8, D_EDGE), lambda i: (i, 0, 0)),
    out_shape=jax.ShapeDtypeStruct((E // 8, 8, D_EDGE), jnp.float32),
)


_RB = 1000


def _dense_body(x_ref, p0_ref, p1_ref, wx_ref, wa_ref, b_ref, o_ref):
    agg = p0_ref[...] + p1_ref[...]
    o_ref[...] = (
        jnp.dot(x_ref[...], wx_ref[...], preferred_element_type=jnp.float32)
        + jnp.dot(agg, wa_ref[...], preferred_element_type=jnp.float32)
        + b_ref[...]
    )


_dense = pl.pallas_call(
    _dense_body,
    grid=(N // _RB,),
    in_specs=[
        pl.BlockSpec((_RB, D_FEAT), lambda i: (i, 0)),
        pl.BlockSpec((_RB, D_EDGE), lambda i: (i, 0)),
        pl.BlockSpec((_RB, D_EDGE), lambda i: (i, 0)),
        pl.BlockSpec((D_FEAT, D_FEAT), lambda i: (0, 0)),
        pl.BlockSpec((D_EDGE, D_FEAT), lambda i: (0, 0)),
        pl.BlockSpec((1, D_FEAT), lambda i: (0, 0)),
    ],
    out_specs=pl.BlockSpec((_RB, D_FEAT), lambda i: (i, 0)),
    out_shape=jax.ShapeDtypeStruct((N, D_FEAT), jnp.float32),
)


def kernel(x, edge_index, edge_attr, pos, W, b):
    recv = edge_index[1]
    packed = _repack(edge_attr.T)
    edge_c = packed.reshape(E, D_EDGE)
    partials = _sc_scatter(edge_c, recv)
    x_ = _dense(
        x,
        partials[0],
        partials[1],
        W[:D_FEAT],
        W[D_FEAT:],
        b.reshape(1, D_FEAT),
    )
    return (x_, edge_attr, edge_index, pos)


# speedup vs baseline: 1.2418x; 1.2418x over previous
import functools

import jax
import jax.numpy as jnp
from jax import lax
from jax.experimental import pallas as pl
from jax.experimental.pallas import tpu as pltpu
from jax.experimental.pallas import tpu_sc as plsc

N = 10000
E = 320000
D_FEAT = 128
D_EDGE = 16

NC = 2
NS = 16
NW = NC * NS
EPW = E // NW
CHUNK = 2000
NCHUNK = EPW // CHUNK
NPAD = 10240
ROWS_PER_TILE = NPAD // NS

_mesh = plsc.VectorSubcoreMesh(
    core_axis_name="c", subcore_axis_name="s", num_cores=NC, num_subcores=NS
)


@functools.partial(
    pl.kernel,
    out_type=jax.ShapeDtypeStruct((NC, NPAD, D_EDGE), jnp.float32),
    mesh=_mesh,
    compiler_params=pltpu.CompilerParams(use_tc_tiling_on_sc=False),
    scratch_types=[
        pltpu.VMEM_SHARED((NPAD, D_EDGE), jnp.float32),
        pltpu.VMEM((EPW,), jnp.int32),
        pltpu.VMEM((CHUNK, D_EDGE), jnp.float32),
        pltpu.VMEM((CHUNK, D_EDGE), jnp.float32),
        pltpu.VMEM((ROWS_PER_TILE, D_EDGE), jnp.float32),
        pltpu.SemaphoreType.DMA,
        pltpu.SemaphoreType.DMA,
    ],
)
def _sc_scatter(edge_hbm, idx_hbm, out_hbm, acc, idx_v, ebufa, ebufb, zbuf, sema, semb):
    cid = lax.axis_index("c")
    sid = lax.axis_index("s")
    wid = sid * NC + cid
    ebase = wid * EPW

    @pl.loop(0, ROWS_PER_TILE)
    def _zero(i):
        zbuf[i, :] = jnp.zeros((D_EDGE,), jnp.float32)

    pltpu.sync_copy(zbuf, acc.at[pl.ds(sid * ROWS_PER_TILE, ROWS_PER_TILE)])
    plsc.subcore_barrier()

    pltpu.sync_copy(idx_hbm.at[pl.ds(ebase, EPW)], idx_v)

    bufs = (ebufa, ebufb)
    sems = (sema, semb)
    cps = [None, None]
    for j in range(NCHUNK + 1):
        if j < NCHUNK:
            cps[j % 2] = pltpu.async_copy(
                edge_hbm.at[pl.ds(ebase + j * CHUNK, CHUNK)], bufs[j % 2], sems[j % 2]
            )
        if j >= 1:
            k = j - 1
            cps[k % 2].wait()
            pltpu.sync_copy(
                bufs[k % 2], acc.at[idx_v.at[pl.ds(k * CHUNK, CHUNK)]], add=True
            )

    plsc.subcore_barrier()

    rows = pl.ds(sid * ROWS_PER_TILE, ROWS_PER_TILE)
    pltpu.sync_copy(acc.at[rows], zbuf)
    pltpu.sync_copy(zbuf, out_hbm.at[cid, rows])


_RB = 1000


def _dense_body(x_ref, p0_ref, p1_ref, wx_ref, wa_ref, b_ref, o_ref):
    agg = p0_ref[...] + p1_ref[...]
    o_ref[...] = (
        jnp.dot(x_ref[...], wx_ref[...], preferred_element_type=jnp.float32)
        + jnp.dot(agg, wa_ref[...], preferred_element_type=jnp.float32)
        + b_ref[...]
    )


_dense = pl.pallas_call(
    _dense_body,
    grid=(N // _RB,),
    in_specs=[
        pl.BlockSpec((_RB, D_FEAT), lambda i: (i, 0)),
        pl.BlockSpec((_RB, D_EDGE), lambda i: (i, 0)),
        pl.BlockSpec((_RB, D_EDGE), lambda i: (i, 0)),
        pl.BlockSpec((D_FEAT, D_FEAT), lambda i: (0, 0)),
        pl.BlockSpec((D_EDGE, D_FEAT), lambda i: (0, 0)),
        pl.BlockSpec((1, D_FEAT), lambda i: (0, 0)),
    ],
    out_specs=pl.BlockSpec((_RB, D_FEAT), lambda i: (i, 0)),
    out_shape=jax.ShapeDtypeStruct((N, D_FEAT), jnp.float32),
)


def kernel(x, edge_index, edge_attr, pos, W, b):
    recv = edge_index[1]
    partials = _sc_scatter(edge_attr, recv)
    x_ = _dense(
        x,
        partials[0],
        partials[1],
        W[:D_FEAT],
        W[D_FEAT:],
        b.reshape(1, D_FEAT),
    )
    return (x_, edge_attr, edge_index, pos)
